# baseline (speedup 1.0000x reference)
import functools

import jax
import jax.numpy as jnp
from jax import lax
from jax.experimental import pallas as pl
from jax.experimental.pallas import tpu as pltpu
from jax.experimental.pallas import tpu_sc as plsc

_CHUNK_FRACS = (1, 3, 3, 1)


def _gather_call(batch: int):
    info = plsc.get_sparse_core_info()
    ns = info.num_subcores
    bpw = batch // ns
    unit = bpw // 8
    sizes = [f * unit for f in _CHUNK_FRACS]
    offs = [sum(sizes[:j]) for j in range(len(sizes))]
    nchunk = len(sizes)
    mesh = plsc.VectorSubcoreMesh(core_axis_name="c", subcore_axis_name="s", num_cores=1)

    @functools.partial(
        pl.kernel,
        mesh=mesh,
        out_type=jax.ShapeDtypeStruct((batch,), jnp.float32),
        scratch_types=[
            pltpu.VMEM((bpw,), jnp.int32),
            pltpu.VMEM((bpw,), jnp.float32),
            pltpu.SemaphoreType.DMA((nchunk,)),
            pltpu.SemaphoreType.DMA((nchunk,)),
            pltpu.SemaphoreType.DMA((nchunk,)),
        ],
    )
    def gather_k(value_hbm, state_hbm, out_hbm, idx_v, vals_v, isem, gsem, wsem):
        base = lax.axis_index("s") * bpw
        loads = [
            pltpu.async_copy(
                state_hbm.at[pl.ds(base + offs[j], sizes[j])],
                idx_v.at[pl.ds(offs[j], sizes[j])],
                isem.at[j],
            )
            for j in range(nchunk)
        ]
        gathers = []
        for j in range(nchunk):
            loads[j].wait()
            gathers.append(
                pltpu.async_copy(
                    value_hbm.at[idx_v.at[pl.ds(offs[j], sizes[j])]],
                    vals_v.at[pl.ds(offs[j], sizes[j])],
                    gsem.at[j],
                )
            )
        writes = []
        for j in range(nchunk):
            gathers[j].wait()
            writes.append(
                pltpu.async_copy(
                    vals_v.at[pl.ds(offs[j], sizes[j])],
                    out_hbm.at[pl.ds(base + offs[j], sizes[j])],
                    wsem.at[j],
                )
            )
        for w in writes:
            w.wait()

    return gather_k


def kernel(state, value):
    state = state.astype(jnp.int32)
    return _gather_call(state.shape[0])(value, state)


# speedup vs baseline: 1.0048x; 1.0048x over previous
import functools

import jax
import jax.numpy as jnp
from jax import lax
from jax.experimental import pallas as pl
from jax.experimental.pallas import tpu as pltpu
from jax.experimental.pallas import tpu_sc as plsc

_CHUNK_FRACS = (2, 2, 2, 2)


def _gather_call(batch: int):
    info = plsc.get_sparse_core_info()
    ns = info.num_subcores
    bpw = batch // ns
    unit = bpw // 8
    sizes = [f * unit for f in _CHUNK_FRACS]
    offs = [sum(sizes[:j]) for j in range(len(sizes))]
    nchunk = len(sizes)
    mesh = plsc.VectorSubcoreMesh(core_axis_name="c", subcore_axis_name="s", num_cores=1)

    @functools.partial(
        pl.kernel,
        mesh=mesh,
        out_type=jax.ShapeDtypeStruct((batch,), jnp.float32),
        scratch_types=[
            pltpu.VMEM((bpw,), jnp.int32),
            pltpu.VMEM((bpw,), jnp.float32),
            pltpu.SemaphoreType.DMA((nchunk,)),
            pltpu.SemaphoreType.DMA((nchunk,)),
            pltpu.SemaphoreType.DMA((nchunk,)),
        ],
    )
    def gather_k(value_hbm, state_hbm, out_hbm, idx_v, vals_v, isem, gsem, wsem):
        base = lax.axis_index("s") * bpw
        loads = [
            pltpu.async_copy(
                state_hbm.at[pl.ds(base + offs[j], sizes[j])],
                idx_v.at[pl.ds(offs[j], sizes[j])],
                isem.at[j],
            )
            for j in range(nchunk)
        ]
        gathers = []
        for j in range(nchunk):
            loads[j].wait()
            gathers.append(
                pltpu.async_copy(
                    value_hbm.at[idx_v.at[pl.ds(offs[j], sizes[j])]],
                    vals_v.at[pl.ds(offs[j], sizes[j])],
                    gsem.at[j],
                )
            )
        writes = []
        for j in range(nchunk):
            gathers[j].wait()
            writes.append(
                pltpu.async_copy(
                    vals_v.at[pl.ds(offs[j], sizes[j])],
                    out_hbm.at[pl.ds(base + offs[j], sizes[j])],
                    wsem.at[j],
                )
            )
        for w in writes:
            w.wait()

    return gather_k


def kernel(state, value):
    state = state.astype(jnp.int32)
    return _gather_call(state.shape[0])(value, state)
